# baseline (speedup 1.0000x reference)
import functools

import numpy as np
import jax
import jax.numpy as jnp
from jax import lax
from jax.experimental import pallas as pl
from jax.experimental.pallas import tpu as pltpu
from jax.experimental.pallas import tpu_sc as plsc

B, H, W, C = 4, 384, 384, 96
HW = H * W
N = B * HW
NW = 32
PIX_PER_TILE = N // NW
P = 96
CHUNKS = PIX_PER_TILE // P
CSUB = C // 16
GROUPS = P // 16
ZROWS = 32

_DELTA = float(np.float32(2.0) / np.float32(383.0))


def _roundbf(v):
    u = plsc.bitcast(v, jnp.int32)
    lsb = lax.shift_right_logical(u, 16) & 1
    r = (u + 32767 + lsb) & jnp.int32(-65536)
    return plsc.bitcast(r, jnp.float32)


def _tile_body(xf, th, out, theta_v,
               idxa0, idxb0, idxc0, idxd0, bufa0, bufb0, bufc0, bufd0,
               wa0, wb0, wc0, wd0, outv0,
               idxa1, idxb1, idxc1, idxd1, bufa1, bufb1, bufc1, bufd1,
               wa1, wb1, wc1, wd1, outv1,
               zbuf, gsem0, gsem1, osem0, osem1):
    set0 = (idxa0, idxb0, idxc0, idxd0, bufa0, bufb0, bufc0, bufd0,
            wa0, wb0, wc0, wd0, outv0, gsem0, osem0)
    set1 = (idxa1, idxb1, idxc1, idxd1, bufa1, bufb1, bufc1, bufd1,
            wa1, wb1, wc1, wd1, outv1, gsem1, osem1)

    cid = lax.axis_index("c")
    sid = lax.axis_index("s")
    wid = sid * 2 + cid

    pltpu.sync_copy(th, theta_v)

    zv = jnp.zeros((16,), jnp.float32)
    for zr in range(ZROWS):
        for s in range(CSUB):
            zbuf[zr, pl.ds(s * 16, 16)] = zv

    def phase1(k, st):
        (idxa, idxb, idxc, idxd, _, _, _, _,
         wa_v, wb_v, wc_v, wd_v, _, _, _) = st
        n0 = (wid + NW * k) * P
        b = n0 // HW
        rowb = b * HW
        tvec = _roundbf(plsc.load_gather(
            theta_v, [b * 6 + lax.broadcasted_iota(jnp.int32, (16,), 0)]))
        t00 = tvec[0]
        t01 = tvec[1]
        t02 = tvec[2]
        t10 = tvec[3]
        t11 = tvec[4]
        t12 = tvec[5]
        h0 = (n0 // W) % H
        w0 = n0 % W
        ys = _roundbf(jnp.full((16,), h0, jnp.int32).astype(jnp.float32)
                      * _DELTA - 1.0)
        allclip = jnp.full((16,), 1, jnp.int32)
        for g in range(GROUPS):
            w_i = w0 + g * 16 + lax.broadcasted_iota(jnp.int32, (16,), 0)
            xs = _roundbf(w_i.astype(jnp.float32) * _DELTA - 1.0)
            gx = t00 * xs + t01 * ys + t02
            gy = t10 * xs + t11 * ys + t12
            px = ((gx + 1.0) * 382.0) * 0.5
            py = ((gy + 1.0) * 382.0) * 0.5
            xt = px.astype(jnp.int32)
            x0 = jnp.where(px < xt.astype(jnp.float32), xt - 1, xt)
            yt = py.astype(jnp.int32)
            y0 = jnp.where(py < yt.astype(jnp.float32), yt - 1, yt)
            x1 = x0 + 1
            y1 = y0 + 1
            x0c = jnp.clip(x0, 0, W - 1)
            x1c = jnp.clip(x1, 0, W - 1)
            y0c = jnp.clip(y0, 0, H - 1)
            y1c = jnp.clip(y1, 0, H - 1)
            clip = jnp.where((x0c == x1c) & (y0c == y1c), 1, 0)
            allclip = allclip & clip
            x0f = x0c.astype(jnp.float32)
            x1f = x1c.astype(jnp.float32)
            y0f = y0c.astype(jnp.float32)
            y1f = y1c.astype(jnp.float32)
            sl = pl.ds(g * 16, 16)
            wa_v[sl] = (x1f - px) * (y1f - py)
            wb_v[sl] = (x1f - px) * (py - y0f)
            wc_v[sl] = (px - x0f) * (y1f - py)
            wd_v[sl] = (px - x0f) * (py - y0f)
            r0 = rowb + y0c * W
            r1 = rowb + y1c * W
            idxa[sl] = r0 + x0c
            idxb[sl] = r1 + x0c
            idxc[sl] = r0 + x1c
            idxd[sl] = r1 + x1c
        return jnp.min(allclip)

    def fire_gathers(st):
        (idxa, idxb, idxc, idxd, bufa, bufb, bufc, bufd,
         _, _, _, _, _, gsem, _) = st
        pltpu.async_copy(xf.at[idxa], bufa, gsem)
        pltpu.async_copy(xf.at[idxb], bufb, gsem)
        pltpu.async_copy(xf.at[idxc], bufc, gsem)
        pltpu.async_copy(xf.at[idxd], bufd, gsem)

    def wait_gathers(st):
        (idxa, idxb, idxc, idxd, bufa, bufb, bufc, bufd,
         _, _, _, _, _, gsem, _) = st
        pltpu.make_async_copy(xf.at[idxa], bufa, gsem).wait()
        pltpu.make_async_copy(xf.at[idxb], bufb, gsem).wait()
        pltpu.make_async_copy(xf.at[idxc], bufc, gsem).wait()
        pltpu.make_async_copy(xf.at[idxd], bufd, gsem).wait()

    def blend(st):
        (_, _, _, _, bufa, bufb, bufc, bufd,
         wa_v, wb_v, wc_v, wd_v, outv, _, _) = st

        def grp(g, c2):
            gsl = pl.ds(g * 16, 16)
            wavec = wa_v[gsl]
            wbvec = wb_v[gsl]
            wcvec = wc_v[gsl]
            wdvec = wd_v[gsl]
            for j in range(16):
                p = g * 16 + j
                was = wavec[j]
                wbs = wbvec[j]
                wcs = wcvec[j]
                wds = wdvec[j]
                for s in range(CSUB):
                    cs = pl.ds(s * 16, 16)
                    outv[p, cs] = ((was * bufa[p, cs] + wbs * bufb[p, cs])
                                   + wcs * bufc[p, cs]) + wds * bufd[p, cs]
            return c2

        lax.fori_loop(0, GROUPS, grp, 0, unroll=False)

    def fire_out(st, k, skip):
        outv, osem = st[12], st[14]
        n0 = (wid + NW * k) * P

        @pl.when(skip == 0)
        def _():
            pltpu.async_copy(outv, out.at[pl.ds(n0, P)], osem)

        @pl.when(skip != 0)
        def _():
            for z in range(P // ZROWS):
                pltpu.async_copy(zbuf, out.at[pl.ds(n0 + z * ZROWS, ZROWS)],
                                 osem)

    def wait_out(st):
        outv, osem = st[12], st[14]
        pltpu.make_async_copy(outv, out.at[pl.ds(0, P)], osem).wait()

    def process(st, k, skip, i):
        @pl.when(i > 0)
        def _():
            wait_out(st)

        @pl.when(skip == 0)
        def _():
            wait_gathers(st)
            blend(st)

        fire_out(st, k, skip)

    def prefetch(st, k):
        skip = phase1(k, st)

        @pl.when((skip == 0) & (k < CHUNKS))
        def _():
            fire_gathers(st)

        return skip

    sk0 = prefetch(set0, 0)
    sk1 = prefetch(set1, 1)

    def body(i, carry):
        s0, s1 = carry
        c0 = 2 * i
        process(set0, c0, s0, i)
        s0n = prefetch(set0, c0 + 2)
        process(set1, c0 + 1, s1, i)
        s1n = prefetch(set1, c0 + 3)
        return (s0n, s1n)

    lax.fori_loop(0, CHUNKS // 2, body, (sk0, sk1), unroll=False)
    wait_out(set0)
    wait_out(set1)


@jax.jit
def kernel(x, theta):
    xf = x.reshape(N, C)
    th = jnp.pad(theta.reshape(-1), (0, 40)).astype(jnp.float32)
    mesh = plsc.VectorSubcoreMesh(core_axis_name="c", subcore_axis_name="s")
    pset = [
        pltpu.VMEM((P,), jnp.int32),
        pltpu.VMEM((P,), jnp.int32),
        pltpu.VMEM((P,), jnp.int32),
        pltpu.VMEM((P,), jnp.int32),
        pltpu.VMEM((P, C), jnp.float32),
        pltpu.VMEM((P, C), jnp.float32),
        pltpu.VMEM((P, C), jnp.float32),
        pltpu.VMEM((P, C), jnp.float32),
        pltpu.VMEM((P,), jnp.float32),
        pltpu.VMEM((P,), jnp.float32),
        pltpu.VMEM((P,), jnp.float32),
        pltpu.VMEM((P,), jnp.float32),
        pltpu.VMEM((P, C), jnp.float32),
    ]
    run = pl.kernel(
        _tile_body,
        out_type=jax.ShapeDtypeStruct((N, C), jnp.float32),
        mesh=mesh,
        compiler_params=pltpu.CompilerParams(
            use_tc_tiling_on_sc=False,
            needs_layout_passes=False,
        ),
        scratch_types=(
            [pltpu.VMEM((64,), jnp.float32)]
            + pset + pset
            + [pltpu.VMEM((ZROWS, C), jnp.float32),
               pltpu.SemaphoreType.DMA,
               pltpu.SemaphoreType.DMA,
               pltpu.SemaphoreType.DMA,
               pltpu.SemaphoreType.DMA]
        ),
    )
    outf = run(xf, th)
    return outf.reshape(B, H, W, C)


# speedup vs baseline: 1.4411x; 1.4411x over previous
import functools

import numpy as np
import jax
import jax.numpy as jnp
from jax import lax
from jax.experimental import pallas as pl
from jax.experimental.pallas import tpu as pltpu
from jax.experimental.pallas import tpu_sc as plsc

B, H, W, C = 4, 384, 384, 96
HW = H * W
N = B * HW
NW = 32
PIX_PER_TILE = N // NW
P = 96
CHUNKS = PIX_PER_TILE // P
CSUB = C // 16
GROUPS = P // 16
ZROWS = 32

_DELTA = float(np.float32(2.0) / np.float32(383.0))


def _roundbf(v):
    u = plsc.bitcast(v, jnp.int32)
    lsb = lax.shift_right_logical(u, 16) & 1
    r = (u + 32767 + lsb) & jnp.int32(-65536)
    return plsc.bitcast(r, jnp.float32)


def _tile_body(xf, th, out, theta_v,
               idxa0, idxb0, idxc0, idxd0, bufa0, bufb0, bufc0, bufd0,
               wa0, wb0, wc0, wd0, outv0,
               idxa1, idxb1, idxc1, idxd1, bufa1, bufb1, bufc1, bufd1,
               wa1, wb1, wc1, wd1, outv1,
               zbuf, gsem0, gsem1, osem0, osem1):
    set0 = (idxa0, idxb0, idxc0, idxd0, bufa0, bufb0, bufc0, bufd0,
            wa0, wb0, wc0, wd0, outv0, gsem0, osem0)
    set1 = (idxa1, idxb1, idxc1, idxd1, bufa1, bufb1, bufc1, bufd1,
            wa1, wb1, wc1, wd1, outv1, gsem1, osem1)

    cid = lax.axis_index("c")
    sid = lax.axis_index("s")
    wid = sid * 2 + cid

    pltpu.sync_copy(th, theta_v)

    zv = jnp.zeros((16,), jnp.float32)
    for zr in range(ZROWS):
        for s in range(CSUB):
            zbuf[zr, pl.ds(s * 16, 16)] = zv

    def phase1(k, st):
        (idxa, idxb, idxc, idxd, _, _, _, _,
         wa_v, wb_v, wc_v, wd_v, _, _, _) = st
        n0 = (wid * CHUNKS + k) * P
        b = n0 // HW
        rowb = b * HW
        tvec = _roundbf(plsc.load_gather(
            theta_v, [b * 6 + lax.broadcasted_iota(jnp.int32, (16,), 0)]))
        t00 = tvec[0]
        t01 = tvec[1]
        t02 = tvec[2]
        t10 = tvec[3]
        t11 = tvec[4]
        t12 = tvec[5]
        h0 = (n0 // W) % H
        w0 = n0 % W
        ys = _roundbf(jnp.full((16,), h0, jnp.int32).astype(jnp.float32)
                      * _DELTA - 1.0)
        allclip = jnp.full((16,), 1, jnp.int32)
        for g in range(GROUPS):
            w_i = w0 + g * 16 + lax.broadcasted_iota(jnp.int32, (16,), 0)
            xs = _roundbf(w_i.astype(jnp.float32) * _DELTA - 1.0)
            gx = t00 * xs + t01 * ys + t02
            gy = t10 * xs + t11 * ys + t12
            px = ((gx + 1.0) * 382.0) * 0.5
            py = ((gy + 1.0) * 382.0) * 0.5
            xt = px.astype(jnp.int32)
            x0 = jnp.where(px < xt.astype(jnp.float32), xt - 1, xt)
            yt = py.astype(jnp.int32)
            y0 = jnp.where(py < yt.astype(jnp.float32), yt - 1, yt)
            x1 = x0 + 1
            y1 = y0 + 1
            x0c = jnp.clip(x0, 0, W - 1)
            x1c = jnp.clip(x1, 0, W - 1)
            y0c = jnp.clip(y0, 0, H - 1)
            y1c = jnp.clip(y1, 0, H - 1)
            clip = jnp.where((x0c == x1c) & (y0c == y1c), 1, 0)
            allclip = allclip & clip
            x0f = x0c.astype(jnp.float32)
            x1f = x1c.astype(jnp.float32)
            y0f = y0c.astype(jnp.float32)
            y1f = y1c.astype(jnp.float32)
            sl = pl.ds(g * 16, 16)
            wa_v[sl] = (x1f - px) * (y1f - py)
            wb_v[sl] = (x1f - px) * (py - y0f)
            wc_v[sl] = (px - x0f) * (y1f - py)
            wd_v[sl] = (px - x0f) * (py - y0f)
            r0 = rowb + y0c * W
            r1 = rowb + y1c * W
            idxa[sl] = r0 + x0c
            idxb[sl] = r1 + x0c
            idxc[sl] = r0 + x1c
            idxd[sl] = r1 + x1c
        return jnp.min(allclip)

    def fire_gathers(st):
        (idxa, idxb, idxc, idxd, bufa, bufb, bufc, bufd,
         _, _, _, _, _, gsem, _) = st
        pltpu.async_copy(xf.at[idxa], bufa, gsem)
        pltpu.async_copy(xf.at[idxb], bufb, gsem)
        pltpu.async_copy(xf.at[idxc], bufc, gsem)
        pltpu.async_copy(xf.at[idxd], bufd, gsem)

    def wait_gathers(st):
        (idxa, idxb, idxc, idxd, bufa, bufb, bufc, bufd,
         _, _, _, _, _, gsem, _) = st
        pltpu.make_async_copy(xf.at[idxa], bufa, gsem).wait()
        pltpu.make_async_copy(xf.at[idxb], bufb, gsem).wait()
        pltpu.make_async_copy(xf.at[idxc], bufc, gsem).wait()
        pltpu.make_async_copy(xf.at[idxd], bufd, gsem).wait()

    def blend(st):
        (_, _, _, _, bufa, bufb, bufc, bufd,
         wa_v, wb_v, wc_v, wd_v, outv, _, _) = st

        def grp(g, c2):
            gsl = pl.ds(g * 16, 16)
            wavec = wa_v[gsl]
            wbvec = wb_v[gsl]
            wcvec = wc_v[gsl]
            wdvec = wd_v[gsl]
            for j in range(16):
                p = g * 16 + j
                was = wavec[j]
                wbs = wbvec[j]
                wcs = wcvec[j]
                wds = wdvec[j]
                for s in range(CSUB):
                    cs = pl.ds(s * 16, 16)
                    outv[p, cs] = ((was * bufa[p, cs] + wbs * bufb[p, cs])
                                   + wcs * bufc[p, cs]) + wds * bufd[p, cs]
            return c2

        lax.fori_loop(0, GROUPS, grp, 0, unroll=False)

    def fire_out(st, k, skip):
        outv, osem = st[12], st[14]
        n0 = (wid * CHUNKS + k) * P

        @pl.when(skip == 0)
        def _():
            pltpu.async_copy(outv, out.at[pl.ds(n0, P)], osem)

        @pl.when(skip != 0)
        def _():
            for z in range(P // ZROWS):
                pltpu.async_copy(zbuf, out.at[pl.ds(n0 + z * ZROWS, ZROWS)],
                                 osem)

    def wait_out(st):
        outv, osem = st[12], st[14]
        pltpu.make_async_copy(outv, out.at[pl.ds(0, P)], osem).wait()

    def process(st, k, skip, i):
        @pl.when(i > 0)
        def _():
            wait_out(st)

        @pl.when(skip == 0)
        def _():
            wait_gathers(st)
            blend(st)

        fire_out(st, k, skip)

    def prefetch(st, k):
        skip = phase1(k, st)

        @pl.when((skip == 0) & (k < CHUNKS))
        def _():
            fire_gathers(st)

        return skip

    sk0 = prefetch(set0, 0)
    sk1 = prefetch(set1, 1)

    def body(i, carry):
        s0, s1 = carry
        c0 = 2 * i
        process(set0, c0, s0, i)
        s0n = prefetch(set0, c0 + 2)
        process(set1, c0 + 1, s1, i)
        s1n = prefetch(set1, c0 + 3)
        return (s0n, s1n)

    lax.fori_loop(0, CHUNKS // 2, body, (sk0, sk1), unroll=False)
    wait_out(set0)
    wait_out(set1)


@jax.jit
def kernel(x, theta):
    xf = x.reshape(N, C)
    th = jnp.pad(theta.reshape(-1), (0, 40)).astype(jnp.float32)
    mesh = plsc.VectorSubcoreMesh(core_axis_name="c", subcore_axis_name="s")
    pset = [
        pltpu.VMEM((P,), jnp.int32),
        pltpu.VMEM((P,), jnp.int32),
        pltpu.VMEM((P,), jnp.int32),
        pltpu.VMEM((P,), jnp.int32),
        pltpu.VMEM((P, C), jnp.float32),
        pltpu.VMEM((P, C), jnp.float32),
        pltpu.VMEM((P, C), jnp.float32),
        pltpu.VMEM((P, C), jnp.float32),
        pltpu.VMEM((P,), jnp.float32),
        pltpu.VMEM((P,), jnp.float32),
        pltpu.VMEM((P,), jnp.float32),
        pltpu.VMEM((P,), jnp.float32),
        pltpu.VMEM((P, C), jnp.float32),
    ]
    run = pl.kernel(
        _tile_body,
        out_type=jax.ShapeDtypeStruct((N, C), jnp.float32),
        mesh=mesh,
        compiler_params=pltpu.CompilerParams(
            use_tc_tiling_on_sc=False,
            needs_layout_passes=False,
        ),
        scratch_types=(
            [pltpu.VMEM((64,), jnp.float32)]
            + pset + pset
            + [pltpu.VMEM((ZROWS, C), jnp.float32),
               pltpu.SemaphoreType.DMA,
               pltpu.SemaphoreType.DMA,
               pltpu.SemaphoreType.DMA,
               pltpu.SemaphoreType.DMA]
        ),
    )
    outf = run(xf, th)
    return outf.reshape(B, H, W, C)
